# Initial kernel scaffold; baseline (speedup 1.0000x reference)
import jax
import jax.numpy as jnp
from jax.experimental import pallas as pl


def kernel(hidden_states, W_gate, W1, W2, W3):
    raise NotImplementedError("write your pallas kernel here")


# speedup vs baseline: 1.1137x; 1.1137x over previous
import jax
import jax.numpy as jnp
from jax.experimental import pallas as pl
from jax.experimental.pallas import tpu as pltpu


def _moe_body(ne, bt, bf, x_ref, wg_ref, w1_ref, w2_ref, w3_ref,
              out_ref, logits_ref, wfull_ref):
    e = pl.program_id(0)
    f = pl.program_id(1)
    b = pl.program_id(2)
    rows = pl.ds(b * bt, bt)
    xb = x_ref[rows, :]

    @pl.when((e == 0) & (f == 0))
    def _router():
        logits = jax.lax.dot_general(
            xb, wg_ref[...], (((1,), (1,)), ((), ())),
            preferred_element_type=jnp.float32)
        logits_ref[rows, :] = logits
        m = jnp.max(logits, axis=-1, keepdims=True)
        ex = jnp.exp(logits - m)
        scores = ex / jnp.sum(ex, axis=-1, keepdims=True)
        cols = jax.lax.broadcasted_iota(jnp.int32, scores.shape, 1)
        m1 = jnp.max(scores, axis=-1, keepdims=True)
        idx1 = jnp.min(jnp.where(scores == m1, cols, ne), axis=-1,
                       keepdims=True)
        masked = jnp.where(cols == idx1, -jnp.inf, scores)
        m2 = jnp.max(masked, axis=-1, keepdims=True)
        idx2 = jnp.min(jnp.where(masked == m2, cols, ne), axis=-1,
                       keepdims=True)
        denom = jnp.clip(m1 + m2, 1e-12, None)
        wf = (jnp.where(cols == idx1, m1 / denom, 0.0)
              + jnp.where(cols == idx2, m2 / denom, 0.0))
        wfull_ref[rows, :] = wf.astype(jnp.float32)
        out_ref[rows, :] = jnp.zeros_like(xb)

    gate = jax.lax.dot_general(
        xb, w1_ref[0], (((1,), (1,)), ((), ())),
        preferred_element_type=jnp.float32)
    up = jax.lax.dot_general(
        xb, w3_ref[0], (((1,), (1,)), ((), ())),
        preferred_element_type=jnp.float32)
    act = gate * jax.nn.sigmoid(gate) * up
    yp = jax.lax.dot_general(
        act, w2_ref[0], (((1,), (1,)), ((), ())),
        preferred_element_type=jnp.float32)
    w8 = wfull_ref[rows, :]
    cols = jax.lax.broadcasted_iota(jnp.int32, w8.shape, 1)
    we = jnp.sum(jnp.where(cols == e, w8, 0.0), axis=-1, keepdims=True)
    out_ref[rows, :] += yp * we


def kernel(hidden_states, W_gate, W1, W2, W3):
    B, S, D = hidden_states.shape
    T = B * S
    E, F = W1.shape[0], W1.shape[1]
    x = hidden_states.reshape(T, D)

    bt = min(256, T)
    bf = min(1024, F)
    nb = T // bt
    nf = F // bf

    body = lambda *refs: _moe_body(E, bt, bf, *refs)
    final, logits = pl.pallas_call(
        body,
        grid=(E, nf, nb),
        in_specs=[
            pl.BlockSpec((T, D), lambda e, f, b: (0, 0)),
            pl.BlockSpec((E, D), lambda e, f, b: (0, 0)),
            pl.BlockSpec((1, bf, D), lambda e, f, b: (e, f, 0)),
            pl.BlockSpec((1, D, bf), lambda e, f, b: (e, 0, f)),
            pl.BlockSpec((1, bf, D), lambda e, f, b: (e, f, 0)),
        ],
        out_specs=[
            pl.BlockSpec((T, D), lambda e, f, b: (0, 0)),
            pl.BlockSpec((T, E), lambda e, f, b: (0, 0)),
        ],
        out_shape=[
            jax.ShapeDtypeStruct((T, D), jnp.float32),
            jax.ShapeDtypeStruct((T, E), jnp.float32),
        ],
        scratch_shapes=[pltpu.VMEM((T, E), jnp.float32)],
        compiler_params=pltpu.CompilerParams(
            dimension_semantics=("arbitrary", "arbitrary", "arbitrary"),
            vmem_limit_bytes=63 * 1024 * 1024,
        ),
    )(x, W_gate, W1, W2, W3)
    return final.reshape(B, S, D), logits
